# Initial kernel scaffold; baseline (speedup 1.0000x reference)
import jax
import jax.numpy as jnp
from jax.experimental import pallas as pl


def kernel(x, edge_index, global_features, w_emb, b_emb, w_gcn0, b_gcn0, w_gcn1, b_gcn1, w_gcn2, b_gcn2, bn_g0, bn_b0, bn_g1, bn_b1, bn_g2, bn_b2, w_gf, b_gf, w_q1, b_q1, w_q2, b_q2, w_q3, b_q3):
    raise NotImplementedError("write your pallas kernel here")


# speedup vs baseline: 2.7324x; 2.7324x over previous
import dataclasses
import functools

import jax
import jax.numpy as jnp
from jax import lax
from jax.experimental import pallas as pl
from jax.experimental.pallas import tpu as pltpu
from jax.experimental.pallas import tpu_sc as plsc

_N = 100000
_E = 1600000
_H = 128
_A = 30

_NC, _NS = 2, 16
_S = 12544
_B = 8
_NPAD = _S * _B
_SLAB = _S + 128
_TRASH = _S
_CH = 128
_ALIGN = _NS * _CH
_EPAD = _E + _B * _ALIGN
_DHALF = 65536
_DEGPAD = 2 * _DHALF

_RB = 2000
_NG = _N // _RB

def _sc_mesh():
    return plsc.VectorSubcoreMesh(core_axis_name="c", subcore_axis_name="s")


def _sc_params():
    cp = pltpu.CompilerParams()
    if "needs_layout_passes" in pltpu.CompilerParams.__dataclass_fields__:
        cp = dataclasses.replace(cp, needs_layout_passes=False)
    return cp


_EPT = _E // _NS
_DCH = 2000
_HR = _DHALF // 128
_HSTRIPE = _HR // _NS
_MCH = 128


def _deg_kernel_body(dst_hbm, out_hbm, dbuf, hist, idxw, acc, sem):
    c = lax.axis_index("c")
    s = lax.axis_index("s")
    lo = c * _DHALF
    ones = jnp.ones((16,), jnp.float32)
    iota = lax.iota(jnp.int32, 16)

    @pl.loop(0, _HR)
    def _z(j):
        @pl.loop(0, 128, step=16)
        def _zz(k):
            hist[j, pl.ds(k, 16)] = jnp.zeros((16,), jnp.float32)

    pltpu.sync_copy(hist.at[pl.ds(0, _HSTRIPE)], acc.at[pl.ds(s * _HSTRIPE, _HSTRIPE)])
    plsc.subcore_barrier()

    @pl.loop(0, _EPT // _DCH)
    def _chunk(i):
        base = s * _EPT + i * _DCH
        pltpu.sync_copy(dst_hbm.at[pl.ds(base, _DCH)], dbuf)

        @pl.loop(0, _DCH, step=16)
        def _vec(j):
            d = dbuf[pl.ds(j, 16)]
            dl = d - lo
            m = (dl >= 0) & (dl < _DHALF)
            dlc = jnp.clip(dl, 0, _DHALF - 1)
            plsc.addupdate_scatter(hist, [dlc >> 7, dlc & 127], ones, mask=m)

    plsc.subcore_barrier()

    @pl.loop(0, _HR // _MCH)
    def _merge(r):
        @pl.loop(0, _MCH, step=16)
        def _ix(k):
            idxw[pl.ds(k, 16)] = iota + (r * _MCH + k)
        pltpu.sync_copy(hist.at[pl.ds(r * _MCH, _MCH)], acc.at[idxw], add=True)

    plsc.subcore_barrier()
    pltpu.sync_copy(acc.at[pl.ds(s * _HSTRIPE, _HSTRIPE)],
                    out_hbm.at[pl.ds(c * _HR + s * _HSTRIPE, _HSTRIPE)])


def _deg_sc(dst):
    k = functools.partial(
        pl.kernel,
        mesh=_sc_mesh(),
        compiler_params=_sc_params(),
        out_type=jax.ShapeDtypeStruct((_DEGPAD // 128, 128), jnp.float32),
        scratch_types=[
            pltpu.VMEM((_DCH,), jnp.int32),
            pltpu.VMEM((_HR, 128), jnp.float32),
            pltpu.VMEM((_MCH,), jnp.int32),
            pltpu.VMEM_SHARED((_HR, 128), jnp.float32),
            pltpu.SemaphoreType.DMA,
        ],
    )(_deg_kernel_body)
    return k(dst)


_ZSTRIPE = _SLAB // 16
_WSTRIPE = _S // 16
_ZB = 24


def _seg_kernel_body(hp_hbm, srcs_hbm, dstl_hbm, offs_hbm, out_hbm,
                     src_v, dst_v, rows_v, zbuf, offs_v, slab, sem):
    c = lax.axis_index("c")
    s = lax.axis_index("s")
    iota = lax.iota(jnp.int32, 16)

    pltpu.sync_copy(offs_hbm, offs_v)

    @pl.loop(0, _ZB)
    def _z(j):
        @pl.loop(0, _H, step=16)
        def _zz(k):
            zbuf[j, pl.ds(k, 16)] = jnp.zeros((16,), jnp.float32)

    for r in range(_B // _NC):
        b = _NC * r + c
        ov = offs_v[...]
        zero16 = jnp.zeros((16,), jnp.int32)
        lo = jnp.sum(jnp.where(iota == zero16 + b, ov, zero16))
        hi = jnp.sum(jnp.where(iota == zero16 + (b + 1), ov, zero16))
        cnt_sub = (hi - lo) // _NS

        zb = s * _ZSTRIPE
        for t in range(_ZSTRIPE // _ZB):
            pltpu.sync_copy(zbuf, slab.at[pl.ds(zb + t * _ZB, _ZB)])
        plsc.subcore_barrier()

        e0 = lo + s * cnt_sub

        def _chunk(i, carry):
            base = pl.multiple_of(e0 + i * _CH, _CH)
            pltpu.sync_copy(srcs_hbm.at[pl.ds(base, _CH)], src_v)
            pltpu.async_copy(hp_hbm.at[src_v], rows_v, sem).wait()
            pltpu.sync_copy(dstl_hbm.at[pl.ds(base, _CH)], dst_v)
            pltpu.sync_copy(rows_v, slab.at[dst_v], add=True)
            return carry

        lax.fori_loop(0, cnt_sub // _CH, _chunk, 0)
        plsc.subcore_barrier()

        sb = s * _WSTRIPE
        ob = b * _S + s * _WSTRIPE
        for t in range(_WSTRIPE // _CH):
            pltpu.sync_copy(slab.at[pl.ds(sb + t * _CH, _CH)],
                            out_hbm.at[pl.ds(ob + t * _CH, _CH)])
        pltpu.sync_copy(slab.at[pl.ds(sb + (_WSTRIPE // _CH) * _CH, _WSTRIPE % _CH)],
                        out_hbm.at[pl.ds(ob + (_WSTRIPE // _CH) * _CH, _WSTRIPE % _CH)])
        plsc.subcore_barrier()


def _seg_sc(hp, srcs, dstl, offs):
    k = functools.partial(
        pl.kernel,
        mesh=_sc_mesh(),
        compiler_params=_sc_params(),
        out_type=jax.ShapeDtypeStruct((_NPAD, _H), jnp.float32),
        scratch_types=[
            pltpu.VMEM((_CH,), jnp.int32),
            pltpu.VMEM((_CH,), jnp.int32),
            pltpu.VMEM((_CH, _H), jnp.float32),
            pltpu.VMEM((_ZB, _H), jnp.float32),
            pltpu.VMEM((16,), jnp.int32),
            pltpu.VMEM_SHARED((_SLAB, _H), jnp.float32),
            pltpu.SemaphoreType.DMA,
        ],
    )(_seg_kernel_body)
    return k(hp, srcs, dstl, offs)


def _bucketize(src, dst):
    bkt = dst // _S
    cnt = jnp.zeros((_B,), jnp.int32).at[bkt].add(1)
    cntp = ((cnt + (_ALIGN - 1)) // _ALIGN) * _ALIGN
    offs = jnp.concatenate([jnp.zeros((1,), jnp.int32), jnp.cumsum(cntp, dtype=jnp.int32)])
    rank = jnp.zeros((_E,), jnp.int32)
    for k in range(_B):
        mk = bkt == k
        rank = rank + jnp.where(mk, jnp.cumsum(mk.astype(jnp.int32)) - 1, 0)
    pos = offs[bkt] + rank
    srcs = jnp.zeros((_EPAD,), jnp.int32).at[pos].set(src)
    dstl = jnp.full((_EPAD,), _TRASH, jnp.int32).at[pos].set(dst - bkt * _S)
    offs16 = jnp.zeros((16,), jnp.int32).at[: _B + 1].set(offs)
    return srcs, dstl, offs16


_PREC = lax.Precision.HIGHEST


def _dot(a, b):
    return jnp.dot(a, b, preferred_element_type=jnp.float32, precision=_PREC)


def _k1_body(x_ref, deg_ref, wemb_ref, bemb_ref, w0_ref, hp0_ref, dinv_ref):
    h0 = jnp.maximum(_dot(x_ref[...], wemb_ref[...]) + bemb_ref[...], 0.0)
    dinv = lax.rsqrt(deg_ref[...] + 1.0)
    hp0_ref[...] = dinv * _dot(h0, w0_ref[...])
    dinv_ref[...] = dinv


def _k1(xpad, deg2d, wembp, bemb, w0):
    return pl.pallas_call(
        _k1_body,
        grid=(_NG,),
        in_specs=[
            pl.BlockSpec((_RB, 16), lambda i: (i, 0)),
            pl.BlockSpec((_RB, 1), lambda i: (i, 0)),
            pl.BlockSpec((16, _H), lambda i: (0, 0)),
            pl.BlockSpec((1, _H), lambda i: (0, 0)),
            pl.BlockSpec((_H, _H), lambda i: (0, 0)),
        ],
        out_specs=[
            pl.BlockSpec((_RB, _H), lambda i: (i, 0)),
            pl.BlockSpec((_RB, 1), lambda i: (i, 0)),
        ],
        out_shape=[
            jax.ShapeDtypeStruct((_N, _H), jnp.float32),
            jax.ShapeDtypeStruct((_N, 1), jnp.float32),
        ],
    )(xpad, deg2d, wembp, bemb, w0)


def _k2_body(agg_ref, hp_ref, dinv_ref, out_ref):
    i = pl.program_id(0)
    t = dinv_ref[...] * (agg_ref[...] + hp_ref[...])
    s1 = jnp.sum(t, axis=0, keepdims=True)
    s2 = jnp.sum(t * t, axis=0, keepdims=True)

    @pl.when(i == 0)
    def _():
        out_ref[...] = jnp.zeros_like(out_ref)

    out_ref[0:1, :] += s1
    out_ref[1:2, :] += s2

    @pl.when(i == _NG - 1)
    def _():
        m = out_ref[0:1, :] / _N
        out_ref[0:1, :] = m
        out_ref[1:2, :] = out_ref[1:2, :] / _N - m * m


def _k2(agg, hp, dinv):
    return pl.pallas_call(
        _k2_body,
        grid=(_NG,),
        in_specs=[
            pl.BlockSpec((_RB, _H), lambda i: (i, 0)),
            pl.BlockSpec((_RB, _H), lambda i: (i, 0)),
            pl.BlockSpec((_RB, 1), lambda i: (i, 0)),
        ],
        out_specs=pl.BlockSpec((2, _H), lambda i: (0, 0)),
        out_shape=jax.ShapeDtypeStruct((2, _H), jnp.float32),
    )(agg, hp, dinv)


def _bn_relu(agg_blk, hp_blk, dinv_blk, stats_ref, g_ref, bb_ref):
    t = dinv_blk * (agg_blk + hp_blk)
    m = stats_ref[0:1, :]
    v = stats_ref[1:2, :]
    scale = g_ref[...] * lax.rsqrt(v + 1e-5)
    return jnp.maximum((t - m) * scale + bb_ref[...], 0.0)


def _k3_body(agg_ref, hp_ref, dinv_ref, stats_ref, g_ref, bb_ref, res_ref,
             wn_ref, h_ref, hpn_ref, *, residual):
    hn = _bn_relu(agg_ref[...], hp_ref[...], dinv_ref[...], stats_ref, g_ref, bb_ref)
    if residual:
        hn = hn + res_ref[...]
    h_ref[...] = hn
    hpn_ref[...] = dinv_ref[...] * _dot(hn, wn_ref[...])


def _k3(agg, hp, dinv, stats, g, bb, res, wnext, residual):
    body = functools.partial(_k3_body, residual=residual)
    return pl.pallas_call(
        body,
        grid=(_NG,),
        in_specs=[
            pl.BlockSpec((_RB, _H), lambda i: (i, 0)),
            pl.BlockSpec((_RB, _H), lambda i: (i, 0)),
            pl.BlockSpec((_RB, 1), lambda i: (i, 0)),
            pl.BlockSpec((2, _H), lambda i: (0, 0)),
            pl.BlockSpec((1, _H), lambda i: (0, 0)),
            pl.BlockSpec((1, _H), lambda i: (0, 0)),
            pl.BlockSpec((_RB, _H), lambda i: (i, 0)),
            pl.BlockSpec((_H, _H), lambda i: (0, 0)),
        ],
        out_specs=[
            pl.BlockSpec((_RB, _H), lambda i: (i, 0)),
            pl.BlockSpec((_RB, _H), lambda i: (i, 0)),
        ],
        out_shape=[
            jax.ShapeDtypeStruct((_N, _H), jnp.float32),
            jax.ShapeDtypeStruct((_N, _H), jnp.float32),
        ],
    )(agg, hp, dinv, stats, g, bb, res, wnext)


def _k3f_body(agg_ref, hp_ref, dinv_ref, stats_ref, g_ref, bb_ref, res_ref,
              gfp_ref, wgfp_ref, bgf_ref, w1a_ref, w1b_ref, w1c_ref, b1_ref,
              w2_ref, b2_ref, w3_ref, b3_ref, q_ref, psum_ref, pmax_ref):
    i = pl.program_id(0)
    hn = _bn_relu(agg_ref[...], hp_ref[...], dinv_ref[...], stats_ref, g_ref, bb_ref)
    hn = hn + res_ref[...]

    @pl.when(i == 0)
    def _():
        psum_ref[...] = jnp.zeros_like(psum_ref)
        pmax_ref[...] = jnp.full_like(pmax_ref, -jnp.inf)

    psum_ref[...] += jnp.sum(hn, axis=0, keepdims=True)
    pmax_ref[...] = jnp.maximum(pmax_ref[...], jnp.max(hn, axis=0, keepdims=True))

    @pl.when(i == _NG - 1)
    def _():
        hm = psum_ref[...] / _N
        hx = pmax_ref[...]
        gfh = jnp.maximum(_dot(gfp_ref[...], wgfp_ref[...]) + bgf_ref[...], 0.0)
        z1 = jnp.maximum(_dot(hm, w1a_ref[...]) + _dot(hx, w1b_ref[...])
                         + _dot(gfh, w1c_ref[...]) + b1_ref[...], 0.0)
        z2 = jnp.maximum(_dot(z1, w2_ref[...]) + b2_ref[...], 0.0)
        q_ref[...] = _dot(z2, w3_ref[...]) + b3_ref[...]


def _k3f(agg, hp, dinv, stats, g, bb, res, gfp, wgfp, bgf,
         w1a, w1b, w1c, b1, w2, b2, w3, b3):
    full = lambda shape: pl.BlockSpec(shape, lambda i: (0, 0))
    return pl.pallas_call(
        _k3f_body,
        grid=(_NG,),
        in_specs=[
            pl.BlockSpec((_RB, _H), lambda i: (i, 0)),
            pl.BlockSpec((_RB, _H), lambda i: (i, 0)),
            pl.BlockSpec((_RB, 1), lambda i: (i, 0)),
            full((2, _H)),
            full((1, _H)),
            full((1, _H)),
            pl.BlockSpec((_RB, _H), lambda i: (i, 0)),
            full((1, 8)),
            full((8, _H)),
            full((1, _H)),
            full((_H, 2 * _H)),
            full((_H, 2 * _H)),
            full((_H, 2 * _H)),
            full((1, 2 * _H)),
            full((2 * _H, _H)),
            full((1, _H)),
            full((_H, _A)),
            full((1, _A)),
        ],
        out_specs=pl.BlockSpec((1, _A), lambda i: (0, 0)),
        out_shape=jax.ShapeDtypeStruct((1, _A), jnp.float32),
        scratch_shapes=[
            pltpu.VMEM((1, _H), jnp.float32),
            pltpu.VMEM((1, _H), jnp.float32),
        ],
    )(agg, hp, dinv, stats, g, bb, res, gfp, wgfp, bgf,
      w1a, w1b, w1c, b1, w2, b2, w3, b3)


_SEG = _seg_sc
_DEG = _deg_sc


def kernel(x, edge_index, global_features, w_emb, b_emb, w_gcn0, b_gcn0,
           w_gcn1, b_gcn1, w_gcn2, b_gcn2, bn_g0, bn_b0, bn_g1, bn_b1,
           bn_g2, bn_b2, w_gf, b_gf, w_q1, b_q1, w_q2, b_q2, w_q3, b_q3):
    src = edge_index[0]
    dst = edge_index[1]

    srcs, dstl, offs16 = _bucketize(src, dst)
    deg = _DEG(dst)
    deg2d = deg.reshape(_DEGPAD, 1)[:_N]

    xpad = jnp.pad(x, ((0, 0), (0, 16 - x.shape[1])))
    wembp = jnp.pad(w_emb, ((0, 16 - w_emb.shape[0]), (0, 0)))
    gfp = jnp.pad(global_features, ((0, 0), (0, 8 - global_features.shape[1])))
    wgfp = jnp.pad(w_gf, ((0, 8 - w_gf.shape[0]), (0, 0)))

    hp0, dinv = _k1(xpad, deg2d, wembp, b_emb.reshape(1, _H), w_gcn0)

    agg0 = _SEG(hp0, srcs, dstl, offs16)
    stats0 = _k2(agg0, hp0, dinv)
    h1, hp1 = _k3(agg0, hp0, dinv, stats0, bn_g0.reshape(1, _H),
                  bn_b0.reshape(1, _H), hp0, w_gcn1, residual=False)

    agg1 = _SEG(hp1, srcs, dstl, offs16)
    stats1 = _k2(agg1, hp1, dinv)
    h2, hp2 = _k3(agg1, hp1, dinv, stats1, bn_g1.reshape(1, _H),
                  bn_b1.reshape(1, _H), h1, w_gcn2, residual=True)

    agg2 = _SEG(hp2, srcs, dstl, offs16)
    stats2 = _k2(agg2, hp2, dinv)
    q = _k3f(agg2, hp2, dinv, stats2, bn_g2.reshape(1, _H),
             bn_b2.reshape(1, _H), h2, gfp, wgfp, b_gf.reshape(1, _H),
             w_q1[:_H], w_q1[_H:2 * _H], w_q1[2 * _H:], b_q1.reshape(1, 2 * _H),
             w_q2, b_q2.reshape(1, _H), w_q3, b_q3.reshape(1, _A))
    return q


# speedup vs baseline: 8.0974x; 2.9635x over previous
import dataclasses
import functools

import jax
import jax.numpy as jnp
from jax import lax
from jax.experimental import pallas as pl
from jax.experimental.pallas import tpu as pltpu
from jax.experimental.pallas import tpu_sc as plsc

_N = 100000
_E = 1600000
_H = 128
_A = 30

_NC, _NS = 2, 16
_S = 12544
_B = 8
_NPAD = _S * _B
_SLAB = _S + 128
_TRASH = _S
_CH = 128
_EC = 112
_ALIGN = _NS * _CH
_EPAD = _E + _B * _ALIGN
_DHALF = 65536
_DEGPAD = 2 * _DHALF

_RB = 2000
_NG = _N // _RB

def _sc_mesh():
    return plsc.VectorSubcoreMesh(core_axis_name="c", subcore_axis_name="s")


def _sc_params():
    cp = pltpu.CompilerParams()
    if "needs_layout_passes" in pltpu.CompilerParams.__dataclass_fields__:
        cp = dataclasses.replace(cp, needs_layout_passes=False)
    return cp


_EPT = _E // _NS
_DCH = 2000
_HR = _DHALF // 128
_HSTRIPE = _HR // _NS
_MCH = 128


def _deg_kernel_body(dst_hbm, out_hbm, dbuf, hist, idxw, acc, sem):
    c = lax.axis_index("c")
    s = lax.axis_index("s")
    lo = c * _DHALF
    ones = jnp.ones((16,), jnp.float32)
    iota = lax.iota(jnp.int32, 16)

    @pl.loop(0, _HR)
    def _z(j):
        @pl.loop(0, 128, step=16)
        def _zz(k):
            hist[j, pl.ds(k, 16)] = jnp.zeros((16,), jnp.float32)

    pltpu.sync_copy(hist.at[pl.ds(0, _HSTRIPE)], acc.at[pl.ds(s * _HSTRIPE, _HSTRIPE)])
    plsc.subcore_barrier()

    @pl.loop(0, _EPT // _DCH)
    def _chunk(i):
        base = s * _EPT + i * _DCH
        pltpu.sync_copy(dst_hbm.at[pl.ds(base, _DCH)], dbuf)

        @pl.loop(0, _DCH, step=16)
        def _vec(j):
            d = dbuf[pl.ds(j, 16)]
            dl = d - lo
            m = (dl >= 0) & (dl < _DHALF)
            dlc = jnp.clip(dl, 0, _DHALF - 1)
            plsc.addupdate_scatter(hist, [dlc >> 7, dlc & 127], ones, mask=m)

    plsc.subcore_barrier()

    @pl.loop(0, _HR // _MCH)
    def _merge(r):
        @pl.loop(0, _MCH, step=16)
        def _ix(k):
            idxw[pl.ds(k, 16)] = iota + (r * _MCH + k)
        pltpu.sync_copy(hist.at[pl.ds(r * _MCH, _MCH)], acc.at[idxw], add=True)

    plsc.subcore_barrier()
    pltpu.sync_copy(acc.at[pl.ds(s * _HSTRIPE, _HSTRIPE)],
                    out_hbm.at[pl.ds(c * _HR + s * _HSTRIPE, _HSTRIPE)])


def _deg_sc(dst):
    k = functools.partial(
        pl.kernel,
        mesh=_sc_mesh(),
        compiler_params=_sc_params(),
        out_type=jax.ShapeDtypeStruct((_DEGPAD // 128, 128), jnp.float32),
        scratch_types=[
            pltpu.VMEM((_DCH,), jnp.int32),
            pltpu.VMEM((_HR, 128), jnp.float32),
            pltpu.VMEM((_MCH,), jnp.int32),
            pltpu.VMEM_SHARED((_HR, 128), jnp.float32),
            pltpu.SemaphoreType.DMA,
        ],
    )(_deg_kernel_body)
    return k(dst)


_ZSTRIPE = _SLAB // 16
_WSTRIPE = _S // 16
_ZB = 24


def _seg_kernel_body(hp_hbm, srcs_hbm, dstl_hbm, nch_hbm, out_hbm,
                     src_v, dst_v, rows_v, zbuf, ncv, slab, sem):
    c = lax.axis_index("c")
    s = lax.axis_index("s")
    iota = lax.iota(jnp.int32, 16)
    zero16 = jnp.zeros((16,), jnp.int32)

    @pl.loop(0, _ZB)
    def _z(j):
        @pl.loop(0, _H, step=16)
        def _zz(k):
            zbuf[j, pl.ds(k, 16)] = jnp.zeros((16,), jnp.float32)

    for r in range(_B // _NC):
        b = _NC * r + c

        zb = s * _ZSTRIPE
        for t in range(_ZSTRIPE // _ZB):
            pltpu.sync_copy(zbuf, slab.at[pl.ds(zb + t * _ZB, _ZB)])
        plsc.subcore_barrier()

        for wseg in range(2):
            w = 2 * s + wseg
            pltpu.sync_copy(nch_hbm.at[pl.ds(w * 8, 1)], ncv)
            row = ncv[0, pl.ds(0, 16)]
            nchunks = jnp.sum(jnp.where(iota == zero16 + b, row, zero16))
            rbase = (w * _B + b) * _CAP

            def _chunk(i, carry):
                base = pl.multiple_of(rbase + i * _EC, 16)
                pltpu.sync_copy(srcs_hbm.at[pl.ds(base, _EC)], src_v)
                pltpu.async_copy(hp_hbm.at[src_v], rows_v, sem).wait()
                pltpu.sync_copy(dstl_hbm.at[pl.ds(base, _EC)], dst_v)
                pltpu.sync_copy(rows_v, slab.at[dst_v], add=True)
                return carry

            lax.fori_loop(0, nchunks, _chunk, 0)
        plsc.subcore_barrier()

        sb = s * _WSTRIPE
        ob = b * _S + s * _WSTRIPE
        for t in range(_WSTRIPE // _CH):
            pltpu.sync_copy(slab.at[pl.ds(sb + t * _CH, _CH)],
                            out_hbm.at[pl.ds(ob + t * _CH, _CH)])
        pltpu.sync_copy(slab.at[pl.ds(sb + (_WSTRIPE // _CH) * _CH, _WSTRIPE % _CH)],
                        out_hbm.at[pl.ds(ob + (_WSTRIPE // _CH) * _CH, _WSTRIPE % _CH)])
        plsc.subcore_barrier()


def _seg_sc(hp, srcs, dstl, nch):
    k = functools.partial(
        pl.kernel,
        mesh=_sc_mesh(),
        compiler_params=_sc_params(),
        out_type=jax.ShapeDtypeStruct((_NPAD, _H), jnp.float32),
        scratch_types=[
            pltpu.VMEM((_EC,), jnp.int32),
            pltpu.VMEM((_EC,), jnp.int32),
            pltpu.VMEM((_EC, _H), jnp.float32),
            pltpu.VMEM((_ZB, _H), jnp.float32),
            pltpu.VMEM((1, 16), jnp.int32),
            pltpu.VMEM_SHARED((_SLAB, _H), jnp.float32),
            pltpu.SemaphoreType.DMA,
        ],
    )(_seg_kernel_body)
    return k(hp, srcs, dstl, nch)


_NW = _NC * _NS
_EPW = _E // _NW
_BCH = 2000
_CAP = 50176
_STG = 128


def _bkt_kernel_body(src_hbm, dst_hbm, srcs_hbm, dstl_hbm, nch_hbm,
                     sbuf, dbuf, stg_s, stg_d, ncbuf, fill, nfl, sem):
    c = lax.axis_index("c")
    s = lax.axis_index("s")
    wid = s * _NC + c
    iota = lax.iota(jnp.int32, 16)

    for b in range(_B):
        fill[b] = 0
        nfl[b] = 0

    @pl.loop(0, _EPW // _BCH)
    def _chunk(i):
        base = wid * _EPW + i * _BCH
        pltpu.sync_copy(src_hbm.at[pl.ds(base, _BCH)], sbuf)
        pltpu.sync_copy(dst_hbm.at[pl.ds(base, _BCH)], dbuf)

        @pl.loop(0, _BCH, step=16)
        def _vec(j):
            d = dbuf[pl.ds(j, 16)]
            sv = sbuf[pl.ds(j, 16)]
            bkt = d // _S
            dl = d - bkt * _S
            for b in range(_B):
                m = bkt == jnp.full((16,), b, jnp.int32)
                f = fill[b]
                p = nfl[b] & 1
                row = b * 2 + p
                plsc.store_compressed(stg_s.at[row, pl.ds(f, 16)], sv, mask=m)
                plsc.store_compressed(stg_d.at[row, pl.ds(f, 16)], dl, mask=m)
                f2 = f + jnp.sum(m.astype(jnp.int32))
                fill[b] = f2

                @pl.when(f2 >= _EC)
                def _flush():
                    ob = pl.multiple_of((wid * _B + b) * _CAP + nfl[b] * _EC, 16)
                    pltpu.sync_copy(stg_s.at[row, pl.ds(0, _EC)], srcs_hbm.at[pl.ds(ob, _EC)])
                    pltpu.sync_copy(stg_d.at[row, pl.ds(0, _EC)], dstl_hbm.at[pl.ds(ob, _EC)])
                    stg_s[b * 2 + (1 - p), pl.ds(0, 16)] = stg_s[row, pl.ds(_EC, 16)]
                    stg_d[b * 2 + (1 - p), pl.ds(0, 16)] = stg_d[row, pl.ds(_EC, 16)]
                    fill[b] = f2 - _EC
                    nfl[b] = nfl[b] + 1

    for b in range(_B):
        f = fill[b]
        row = b * 2 + (nfl[b] & 1)
        for t in range(_EC // 16):
            keep = (t * 16 + iota) < f
            vs = stg_s[row, pl.ds(t * 16, 16)]
            vd = stg_d[row, pl.ds(t * 16, 16)]
            stg_s[row, pl.ds(t * 16, 16)] = jnp.where(keep, vs, jnp.zeros((16,), jnp.int32))
            stg_d[row, pl.ds(t * 16, 16)] = jnp.where(keep, vd, jnp.full((16,), _TRASH, jnp.int32))
        ob = pl.multiple_of((wid * _B + b) * _CAP + nfl[b] * _EC, 16)
        pltpu.sync_copy(stg_s.at[row, pl.ds(0, _EC)], srcs_hbm.at[pl.ds(ob, _EC)])
        pltpu.sync_copy(stg_d.at[row, pl.ds(0, _EC)], dstl_hbm.at[pl.ds(ob, _EC)])
    ncvec = jnp.zeros((16,), jnp.int32)
    for b in range(_B):
        ncvec = jnp.where(iota == jnp.full((16,), b, jnp.int32), nfl[b] + 1, ncvec)
    ncbuf[0, :] = ncvec
    pltpu.sync_copy(ncbuf, nch_hbm.at[pl.ds(wid * 8, 1)])


def _bucketize_sc(src, dst):
    k = functools.partial(
        pl.kernel,
        mesh=_sc_mesh(),
        compiler_params=_sc_params(),
        out_type=[
            jax.ShapeDtypeStruct((_NW * _B * _CAP,), jnp.int32),
            jax.ShapeDtypeStruct((_NW * _B * _CAP,), jnp.int32),
            jax.ShapeDtypeStruct((_NW * 8, 16), jnp.int32),
        ],
        scratch_types=[
            pltpu.VMEM((_BCH,), jnp.int32),
            pltpu.VMEM((_BCH,), jnp.int32),
            pltpu.VMEM((_B * 2, _STG), jnp.int32),
            pltpu.VMEM((_B * 2, _STG), jnp.int32),
            pltpu.VMEM((1, 16), jnp.int32),
            pltpu.SMEM((_B,), jnp.int32),
            pltpu.SMEM((_B,), jnp.int32),
            pltpu.SemaphoreType.DMA,
        ],
    )(_bkt_kernel_body)
    return k(src, dst)


_PREC = lax.Precision.HIGHEST


def _dot(a, b):
    return jnp.dot(a, b, preferred_element_type=jnp.float32, precision=_PREC)


def _k1_body(x_ref, deg_ref, wemb_ref, bemb_ref, w0_ref, hp0_ref, dinv_ref):
    h0 = jnp.maximum(_dot(x_ref[...], wemb_ref[...]) + bemb_ref[...], 0.0)
    dinv = lax.rsqrt(deg_ref[...] + 1.0)
    hp0_ref[...] = dinv * _dot(h0, w0_ref[...])
    dinv_ref[...] = dinv


def _k1(xpad, deg2d, wembp, bemb, w0):
    return pl.pallas_call(
        _k1_body,
        grid=(_NG,),
        in_specs=[
            pl.BlockSpec((_RB, 16), lambda i: (i, 0)),
            pl.BlockSpec((_RB, 1), lambda i: (i, 0)),
            pl.BlockSpec((16, _H), lambda i: (0, 0)),
            pl.BlockSpec((1, _H), lambda i: (0, 0)),
            pl.BlockSpec((_H, _H), lambda i: (0, 0)),
        ],
        out_specs=[
            pl.BlockSpec((_RB, _H), lambda i: (i, 0)),
            pl.BlockSpec((_RB, 1), lambda i: (i, 0)),
        ],
        out_shape=[
            jax.ShapeDtypeStruct((_N, _H), jnp.float32),
            jax.ShapeDtypeStruct((_N, 1), jnp.float32),
        ],
    )(xpad, deg2d, wembp, bemb, w0)


def _k2_body(agg_ref, hp_ref, dinv_ref, out_ref):
    i = pl.program_id(0)
    t = dinv_ref[...] * (agg_ref[...] + hp_ref[...])
    s1 = jnp.sum(t, axis=0, keepdims=True)
    s2 = jnp.sum(t * t, axis=0, keepdims=True)

    @pl.when(i == 0)
    def _():
        out_ref[...] = jnp.zeros_like(out_ref)

    out_ref[0:1, :] += s1
    out_ref[1:2, :] += s2

    @pl.when(i == _NG - 1)
    def _():
        m = out_ref[0:1, :] / _N
        out_ref[0:1, :] = m
        out_ref[1:2, :] = out_ref[1:2, :] / _N - m * m


def _k2(agg, hp, dinv):
    return pl.pallas_call(
        _k2_body,
        grid=(_NG,),
        in_specs=[
            pl.BlockSpec((_RB, _H), lambda i: (i, 0)),
            pl.BlockSpec((_RB, _H), lambda i: (i, 0)),
            pl.BlockSpec((_RB, 1), lambda i: (i, 0)),
        ],
        out_specs=pl.BlockSpec((2, _H), lambda i: (0, 0)),
        out_shape=jax.ShapeDtypeStruct((2, _H), jnp.float32),
    )(agg, hp, dinv)


def _bn_relu(agg_blk, hp_blk, dinv_blk, stats_ref, g_ref, bb_ref):
    t = dinv_blk * (agg_blk + hp_blk)
    m = stats_ref[0:1, :]
    v = stats_ref[1:2, :]
    scale = g_ref[...] * lax.rsqrt(v + 1e-5)
    return jnp.maximum((t - m) * scale + bb_ref[...], 0.0)


def _k3_body(agg_ref, hp_ref, dinv_ref, stats_ref, g_ref, bb_ref, res_ref,
             wn_ref, h_ref, hpn_ref, *, residual):
    hn = _bn_relu(agg_ref[...], hp_ref[...], dinv_ref[...], stats_ref, g_ref, bb_ref)
    if residual:
        hn = hn + res_ref[...]
    h_ref[...] = hn
    hpn_ref[...] = dinv_ref[...] * _dot(hn, wn_ref[...])


def _k3(agg, hp, dinv, stats, g, bb, res, wnext, residual):
    body = functools.partial(_k3_body, residual=residual)
    return pl.pallas_call(
        body,
        grid=(_NG,),
        in_specs=[
            pl.BlockSpec((_RB, _H), lambda i: (i, 0)),
            pl.BlockSpec((_RB, _H), lambda i: (i, 0)),
            pl.BlockSpec((_RB, 1), lambda i: (i, 0)),
            pl.BlockSpec((2, _H), lambda i: (0, 0)),
            pl.BlockSpec((1, _H), lambda i: (0, 0)),
            pl.BlockSpec((1, _H), lambda i: (0, 0)),
            pl.BlockSpec((_RB, _H), lambda i: (i, 0)),
            pl.BlockSpec((_H, _H), lambda i: (0, 0)),
        ],
        out_specs=[
            pl.BlockSpec((_RB, _H), lambda i: (i, 0)),
            pl.BlockSpec((_RB, _H), lambda i: (i, 0)),
        ],
        out_shape=[
            jax.ShapeDtypeStruct((_N, _H), jnp.float32),
            jax.ShapeDtypeStruct((_N, _H), jnp.float32),
        ],
    )(agg, hp, dinv, stats, g, bb, res, wnext)


def _k3f_body(agg_ref, hp_ref, dinv_ref, stats_ref, g_ref, bb_ref, res_ref,
              gfp_ref, wgfp_ref, bgf_ref, w1a_ref, w1b_ref, w1c_ref, b1_ref,
              w2_ref, b2_ref, w3_ref, b3_ref, q_ref, psum_ref, pmax_ref):
    i = pl.program_id(0)
    hn = _bn_relu(agg_ref[...], hp_ref[...], dinv_ref[...], stats_ref, g_ref, bb_ref)
    hn = hn + res_ref[...]

    @pl.when(i == 0)
    def _():
        psum_ref[...] = jnp.zeros_like(psum_ref)
        pmax_ref[...] = jnp.full_like(pmax_ref, -jnp.inf)

    psum_ref[...] += jnp.sum(hn, axis=0, keepdims=True)
    pmax_ref[...] = jnp.maximum(pmax_ref[...], jnp.max(hn, axis=0, keepdims=True))

    @pl.when(i == _NG - 1)
    def _():
        hm = psum_ref[...] / _N
        hx = pmax_ref[...]
        gfh = jnp.maximum(_dot(gfp_ref[...], wgfp_ref[...]) + bgf_ref[...], 0.0)
        z1 = jnp.maximum(_dot(hm, w1a_ref[...]) + _dot(hx, w1b_ref[...])
                         + _dot(gfh, w1c_ref[...]) + b1_ref[...], 0.0)
        z2 = jnp.maximum(_dot(z1, w2_ref[...]) + b2_ref[...], 0.0)
        q_ref[...] = _dot(z2, w3_ref[...]) + b3_ref[...]


def _k3f(agg, hp, dinv, stats, g, bb, res, gfp, wgfp, bgf,
         w1a, w1b, w1c, b1, w2, b2, w3, b3):
    full = lambda shape: pl.BlockSpec(shape, lambda i: (0, 0))
    return pl.pallas_call(
        _k3f_body,
        grid=(_NG,),
        in_specs=[
            pl.BlockSpec((_RB, _H), lambda i: (i, 0)),
            pl.BlockSpec((_RB, _H), lambda i: (i, 0)),
            pl.BlockSpec((_RB, 1), lambda i: (i, 0)),
            full((2, _H)),
            full((1, _H)),
            full((1, _H)),
            pl.BlockSpec((_RB, _H), lambda i: (i, 0)),
            full((1, 8)),
            full((8, _H)),
            full((1, _H)),
            full((_H, 2 * _H)),
            full((_H, 2 * _H)),
            full((_H, 2 * _H)),
            full((1, 2 * _H)),
            full((2 * _H, _H)),
            full((1, _H)),
            full((_H, _A)),
            full((1, _A)),
        ],
        out_specs=pl.BlockSpec((1, _A), lambda i: (0, 0)),
        out_shape=jax.ShapeDtypeStruct((1, _A), jnp.float32),
        scratch_shapes=[
            pltpu.VMEM((1, _H), jnp.float32),
            pltpu.VMEM((1, _H), jnp.float32),
        ],
    )(agg, hp, dinv, stats, g, bb, res, gfp, wgfp, bgf,
      w1a, w1b, w1c, b1, w2, b2, w3, b3)


_SEG = _seg_sc
_DEG = _deg_sc
_BKT = _bucketize_sc


def kernel(x, edge_index, global_features, w_emb, b_emb, w_gcn0, b_gcn0,
           w_gcn1, b_gcn1, w_gcn2, b_gcn2, bn_g0, bn_b0, bn_g1, bn_b1,
           bn_g2, bn_b2, w_gf, b_gf, w_q1, b_q1, w_q2, b_q2, w_q3, b_q3):
    src = edge_index[0]
    dst = edge_index[1]

    srcs, dstl, nch = _BKT(src, dst)
    deg = _DEG(dst)
    deg2d = deg.reshape(_DEGPAD, 1)[:_N]

    xpad = jnp.pad(x, ((0, 0), (0, 16 - x.shape[1])))
    wembp = jnp.pad(w_emb, ((0, 16 - w_emb.shape[0]), (0, 0)))
    gfp = jnp.pad(global_features, ((0, 0), (0, 8 - global_features.shape[1])))
    wgfp = jnp.pad(w_gf, ((0, 8 - w_gf.shape[0]), (0, 0)))

    hp0, dinv = _k1(xpad, deg2d, wembp, b_emb.reshape(1, _H), w_gcn0)

    agg0 = _SEG(hp0, srcs, dstl, nch)
    stats0 = _k2(agg0, hp0, dinv)
    h1, hp1 = _k3(agg0, hp0, dinv, stats0, bn_g0.reshape(1, _H),
                  bn_b0.reshape(1, _H), hp0, w_gcn1, residual=False)

    agg1 = _SEG(hp1, srcs, dstl, nch)
    stats1 = _k2(agg1, hp1, dinv)
    h2, hp2 = _k3(agg1, hp1, dinv, stats1, bn_g1.reshape(1, _H),
                  bn_b1.reshape(1, _H), h1, w_gcn2, residual=True)

    agg2 = _SEG(hp2, srcs, dstl, nch)
    stats2 = _k2(agg2, hp2, dinv)
    q = _k3f(agg2, hp2, dinv, stats2, bn_g2.reshape(1, _H),
             bn_b2.reshape(1, _H), h2, gfp, wgfp, b_gf.reshape(1, _H),
             w_q1[:_H], w_q1[_H:2 * _H], w_q1[2 * _H:], b_q1.reshape(1, 2 * _H),
             w_q2, b_q2.reshape(1, _H), w_q3, b_q3.reshape(1, _A))
    return q


# speedup vs baseline: 9.3148x; 1.1503x over previous
import dataclasses
import functools

import jax
import jax.numpy as jnp
from jax import lax
from jax.experimental import pallas as pl
from jax.experimental.pallas import tpu as pltpu
from jax.experimental.pallas import tpu_sc as plsc

_N = 100000
_E = 1600000
_H = 128
_A = 30

_NC, _NS = 2, 16
_S = 11264
_B = 9
_NPAD = _S * _B
_SLAB = _S + 128
_TRASH = _S
_CH = 128
_EC = 112
_ALIGN = _NS * _CH
_EPAD = _E + _B * _ALIGN
_DHALF = 65536
_DEGPAD = 2 * _DHALF

_RB = 2000
_NG = _N // _RB

def _sc_mesh():
    return plsc.VectorSubcoreMesh(core_axis_name="c", subcore_axis_name="s")


def _sc_params():
    cp = pltpu.CompilerParams()
    if "needs_layout_passes" in pltpu.CompilerParams.__dataclass_fields__:
        cp = dataclasses.replace(cp, needs_layout_passes=False)
    return cp


_EPT = _E // _NS
_DCH = 2000
_HR = _DHALF // 128
_HSTRIPE = _HR // _NS
_MCH = 128


def _deg_kernel_body(dst_hbm, out_hbm, dbuf, hist, idxw, acc, sem):
    c = lax.axis_index("c")
    s = lax.axis_index("s")
    lo = c * _DHALF
    ones = jnp.ones((16,), jnp.float32)
    iota = lax.iota(jnp.int32, 16)

    @pl.loop(0, _HR)
    def _z(j):
        @pl.loop(0, 128, step=16)
        def _zz(k):
            hist[j, pl.ds(k, 16)] = jnp.zeros((16,), jnp.float32)

    pltpu.sync_copy(hist.at[pl.ds(0, _HSTRIPE)], acc.at[pl.ds(s * _HSTRIPE, _HSTRIPE)])
    plsc.subcore_barrier()

    @pl.loop(0, _EPT // _DCH)
    def _chunk(i):
        base = s * _EPT + i * _DCH
        pltpu.sync_copy(dst_hbm.at[pl.ds(base, _DCH)], dbuf)

        @pl.loop(0, _DCH, step=16)
        def _vec(j):
            d = dbuf[pl.ds(j, 16)]
            dl = d - lo
            m = (dl >= 0) & (dl < _DHALF)
            dlc = jnp.clip(dl, 0, _DHALF - 1)
            plsc.addupdate_scatter(hist, [dlc >> 7, dlc & 127], ones, mask=m)

    plsc.subcore_barrier()

    @pl.loop(0, _HR // _MCH)
    def _merge(r):
        @pl.loop(0, _MCH, step=16)
        def _ix(k):
            idxw[pl.ds(k, 16)] = iota + (r * _MCH + k)
        pltpu.sync_copy(hist.at[pl.ds(r * _MCH, _MCH)], acc.at[idxw], add=True)

    plsc.subcore_barrier()
    pltpu.sync_copy(acc.at[pl.ds(s * _HSTRIPE, _HSTRIPE)],
                    out_hbm.at[pl.ds(c * _HR + s * _HSTRIPE, _HSTRIPE)])


def _deg_sc(dst):
    k = functools.partial(
        pl.kernel,
        mesh=_sc_mesh(),
        compiler_params=_sc_params(),
        out_type=jax.ShapeDtypeStruct((_DEGPAD // 128, 128), jnp.float32),
        scratch_types=[
            pltpu.VMEM((_DCH,), jnp.int32),
            pltpu.VMEM((_HR, 128), jnp.float32),
            pltpu.VMEM((_MCH,), jnp.int32),
            pltpu.VMEM_SHARED((_HR, 128), jnp.float32),
            pltpu.SemaphoreType.DMA,
        ],
    )(_deg_kernel_body)
    return k(dst)


_ZSTRIPE = _SLAB // 16
_WSTRIPE = _S // 16
_ZB = 8


def _seg_kernel_body(hp_hbm, srcs_hbm, dstl_hbm, nch_hbm, out_hbm,
                     src_v, dst_v, rows_v, src_v2, dst_v2, rows_v2,
                     zbuf, ncv, slab, sem, sem2):
    c = lax.axis_index("c")
    s = lax.axis_index("s")
    iota = lax.iota(jnp.int32, 16)
    zero16 = jnp.zeros((16,), jnp.int32)

    @pl.loop(0, _ZB)
    def _z(j):
        @pl.loop(0, _H, step=16)
        def _zz(k):
            zbuf[j, pl.ds(k, 16)] = jnp.zeros((16,), jnp.float32)

    for r in range((_B + _NC - 1) // _NC):
        b = _NC * r + c

        @pl.when(b < _B)
        def _round():
            zb = s * _ZSTRIPE
            for t in range(_ZSTRIPE // _ZB):
                pltpu.sync_copy(zbuf, slab.at[pl.ds(zb + t * _ZB, _ZB)])
            plsc.subcore_barrier()

            for wseg in range(2):
                w = 2 * s + wseg
                pltpu.sync_copy(nch_hbm.at[pl.ds(w * 8, 1)], ncv)
                row = ncv[0, pl.ds(0, 16)]
                nchunks = jnp.sum(jnp.where(iota == zero16 + b, row, zero16))
                rbase = (w * _B + b) * _CAP

                def _ld(k, sv, rv, sm):
                    basek = pl.multiple_of(rbase + k * _EC, 16)
                    pltpu.sync_copy(srcs_hbm.at[pl.ds(basek, _EC)], sv)
                    pltpu.async_copy(hp_hbm.at[sv], rv, sm)

                @pl.when(nchunks > 0)
                def _prime():
                    _ld(0, src_v, rows_v, sem)

                bufs = [(0, src_v, dst_v, rows_v, sem, src_v2, rows_v2, sem2),
                        (1, src_v2, dst_v2, rows_v2, sem2, src_v, rows_v, sem)]

                def _chunk(i, carry):
                    for p, sv, dv, rv, sm, svn, rvn, smn in bufs:
                        @pl.when((i & 1) == p)
                        def _go():
                            @pl.when(i + 1 < nchunks)
                            def _pf():
                                _ld(i + 1, svn, rvn, smn)

                            base = pl.multiple_of(rbase + i * _EC, 16)
                            pltpu.sync_copy(dstl_hbm.at[pl.ds(base, _EC)], dv)
                            pltpu.make_async_copy(hp_hbm.at[sv], rv, sm).wait()
                            pltpu.sync_copy(rv, slab.at[dv], add=True)
                    return carry

                lax.fori_loop(0, nchunks, _chunk, 0)
            plsc.subcore_barrier()

            sb = s * _WSTRIPE
            ob = b * _S + s * _WSTRIPE
            for t in range(_WSTRIPE // _CH):
                pltpu.sync_copy(slab.at[pl.ds(sb + t * _CH, _CH)],
                                out_hbm.at[pl.ds(ob + t * _CH, _CH)])
            pltpu.sync_copy(slab.at[pl.ds(sb + (_WSTRIPE // _CH) * _CH, _WSTRIPE % _CH)],
                            out_hbm.at[pl.ds(ob + (_WSTRIPE // _CH) * _CH, _WSTRIPE % _CH)])
            plsc.subcore_barrier()


def _seg_sc(hp, srcs, dstl, nch):
    k = functools.partial(
        pl.kernel,
        mesh=_sc_mesh(),
        compiler_params=_sc_params(),
        out_type=jax.ShapeDtypeStruct((_NPAD, _H), jnp.float32),
        scratch_types=[
            pltpu.VMEM((_EC,), jnp.int32),
            pltpu.VMEM((_EC,), jnp.int32),
            pltpu.VMEM((_EC, _H), jnp.float32),
            pltpu.VMEM((_EC,), jnp.int32),
            pltpu.VMEM((_EC,), jnp.int32),
            pltpu.VMEM((_EC, _H), jnp.float32),
            pltpu.VMEM((_ZB, _H), jnp.float32),
            pltpu.VMEM((1, 16), jnp.int32),
            pltpu.VMEM_SHARED((_SLAB, _H), jnp.float32),
            pltpu.SemaphoreType.DMA,
            pltpu.SemaphoreType.DMA,
        ],
    )(_seg_kernel_body)
    return k(hp, srcs, dstl, nch)


_NW = _NC * _NS
_EPW = _E // _NW
_BCH = 2000
_CAP = 50176
_STG = 128


def _bkt_kernel_body(src_hbm, dst_hbm, srcs_hbm, dstl_hbm, nch_hbm,
                     sbuf, dbuf, stg_s, stg_d, ncbuf, fill, nfl, sem):
    c = lax.axis_index("c")
    s = lax.axis_index("s")
    wid = s * _NC + c
    iota = lax.iota(jnp.int32, 16)

    for b in range(_B):
        fill[b] = 0
        nfl[b] = 0

    @pl.loop(0, _EPW // _BCH)
    def _chunk(i):
        base = wid * _EPW + i * _BCH
        pltpu.sync_copy(src_hbm.at[pl.ds(base, _BCH)], sbuf)
        pltpu.sync_copy(dst_hbm.at[pl.ds(base, _BCH)], dbuf)

        @pl.loop(0, _BCH, step=16)
        def _vec(j):
            d = dbuf[pl.ds(j, 16)]
            sv = sbuf[pl.ds(j, 16)]
            bkt = d // _S
            dl = d - bkt * _S
            for b in range(_B):
                m = bkt == jnp.full((16,), b, jnp.int32)
                f = fill[b]
                p = nfl[b] & 1
                row = b * 2 + p
                plsc.store_compressed(stg_s.at[row, pl.ds(f, 16)], sv, mask=m)
                plsc.store_compressed(stg_d.at[row, pl.ds(f, 16)], dl, mask=m)
                f2 = f + jnp.sum(m.astype(jnp.int32))
                fill[b] = f2

                @pl.when(f2 >= _EC)
                def _flush():
                    ob = pl.multiple_of((wid * _B + b) * _CAP + nfl[b] * _EC, 16)
                    pltpu.sync_copy(stg_s.at[row, pl.ds(0, _EC)], srcs_hbm.at[pl.ds(ob, _EC)])
                    pltpu.sync_copy(stg_d.at[row, pl.ds(0, _EC)], dstl_hbm.at[pl.ds(ob, _EC)])
                    stg_s[b * 2 + (1 - p), pl.ds(0, 16)] = stg_s[row, pl.ds(_EC, 16)]
                    stg_d[b * 2 + (1 - p), pl.ds(0, 16)] = stg_d[row, pl.ds(_EC, 16)]
                    fill[b] = f2 - _EC
                    nfl[b] = nfl[b] + 1

    for b in range(_B):
        f = fill[b]
        row = b * 2 + (nfl[b] & 1)
        for t in range(_EC // 16):
            keep = (t * 16 + iota) < f
            vs = stg_s[row, pl.ds(t * 16, 16)]
            vd = stg_d[row, pl.ds(t * 16, 16)]
            stg_s[row, pl.ds(t * 16, 16)] = jnp.where(keep, vs, jnp.zeros((16,), jnp.int32))
            stg_d[row, pl.ds(t * 16, 16)] = jnp.where(keep, vd, jnp.full((16,), _TRASH, jnp.int32))
        ob = pl.multiple_of((wid * _B + b) * _CAP + nfl[b] * _EC, 16)
        pltpu.sync_copy(stg_s.at[row, pl.ds(0, _EC)], srcs_hbm.at[pl.ds(ob, _EC)])
        pltpu.sync_copy(stg_d.at[row, pl.ds(0, _EC)], dstl_hbm.at[pl.ds(ob, _EC)])
    ncvec = jnp.zeros((16,), jnp.int32)
    for b in range(_B):
        ncvec = jnp.where(iota == jnp.full((16,), b, jnp.int32), nfl[b] + 1, ncvec)
    ncbuf[0, :] = ncvec
    pltpu.sync_copy(ncbuf, nch_hbm.at[pl.ds(wid * 8, 1)])


def _bucketize_sc(src, dst):
    k = functools.partial(
        pl.kernel,
        mesh=_sc_mesh(),
        compiler_params=_sc_params(),
        out_type=[
            jax.ShapeDtypeStruct((_NW * _B * _CAP,), jnp.int32),
            jax.ShapeDtypeStruct((_NW * _B * _CAP,), jnp.int32),
            jax.ShapeDtypeStruct((_NW * 8, 16), jnp.int32),
        ],
        scratch_types=[
            pltpu.VMEM((_BCH,), jnp.int32),
            pltpu.VMEM((_BCH,), jnp.int32),
            pltpu.VMEM((_B * 2, _STG), jnp.int32),
            pltpu.VMEM((_B * 2, _STG), jnp.int32),
            pltpu.VMEM((1, 16), jnp.int32),
            pltpu.SMEM((_B,), jnp.int32),
            pltpu.SMEM((_B,), jnp.int32),
            pltpu.SemaphoreType.DMA,
        ],
    )(_bkt_kernel_body)
    return k(src, dst)


_PREC = lax.Precision.HIGHEST


def _dot(a, b):
    return jnp.dot(a, b, preferred_element_type=jnp.float32, precision=_PREC)


def _k1_body(x_ref, deg_ref, wemb_ref, bemb_ref, w0_ref, hp0_ref, dinv_ref):
    h0 = jnp.maximum(_dot(x_ref[...], wemb_ref[...]) + bemb_ref[...], 0.0)
    dinv = lax.rsqrt(deg_ref[...] + 1.0)
    hp0_ref[...] = dinv * _dot(h0, w0_ref[...])
    dinv_ref[...] = dinv


def _k1(xpad, deg2d, wembp, bemb, w0):
    return pl.pallas_call(
        _k1_body,
        grid=(_NG,),
        in_specs=[
            pl.BlockSpec((_RB, 16), lambda i: (i, 0)),
            pl.BlockSpec((_RB, 1), lambda i: (i, 0)),
            pl.BlockSpec((16, _H), lambda i: (0, 0)),
            pl.BlockSpec((1, _H), lambda i: (0, 0)),
            pl.BlockSpec((_H, _H), lambda i: (0, 0)),
        ],
        out_specs=[
            pl.BlockSpec((_RB, _H), lambda i: (i, 0)),
            pl.BlockSpec((_RB, 1), lambda i: (i, 0)),
        ],
        out_shape=[
            jax.ShapeDtypeStruct((_N, _H), jnp.float32),
            jax.ShapeDtypeStruct((_N, 1), jnp.float32),
        ],
    )(xpad, deg2d, wembp, bemb, w0)


def _k2_body(agg_ref, hp_ref, dinv_ref, out_ref):
    i = pl.program_id(0)
    t = dinv_ref[...] * (agg_ref[...] + hp_ref[...])
    s1 = jnp.sum(t, axis=0, keepdims=True)
    s2 = jnp.sum(t * t, axis=0, keepdims=True)

    @pl.when(i == 0)
    def _():
        out_ref[...] = jnp.zeros_like(out_ref)

    out_ref[0:1, :] += s1
    out_ref[1:2, :] += s2

    @pl.when(i == _NG - 1)
    def _():
        m = out_ref[0:1, :] / _N
        out_ref[0:1, :] = m
        out_ref[1:2, :] = out_ref[1:2, :] / _N - m * m


def _k2(agg, hp, dinv):
    return pl.pallas_call(
        _k2_body,
        grid=(_NG,),
        in_specs=[
            pl.BlockSpec((_RB, _H), lambda i: (i, 0)),
            pl.BlockSpec((_RB, _H), lambda i: (i, 0)),
            pl.BlockSpec((_RB, 1), lambda i: (i, 0)),
        ],
        out_specs=pl.BlockSpec((2, _H), lambda i: (0, 0)),
        out_shape=jax.ShapeDtypeStruct((2, _H), jnp.float32),
    )(agg, hp, dinv)


def _bn_relu(agg_blk, hp_blk, dinv_blk, stats_ref, g_ref, bb_ref):
    t = dinv_blk * (agg_blk + hp_blk)
    m = stats_ref[0:1, :]
    v = stats_ref[1:2, :]
    scale = g_ref[...] * lax.rsqrt(v + 1e-5)
    return jnp.maximum((t - m) * scale + bb_ref[...], 0.0)


def _k3_body(agg_ref, hp_ref, dinv_ref, stats_ref, g_ref, bb_ref, res_ref,
             wn_ref, h_ref, hpn_ref, *, residual):
    hn = _bn_relu(agg_ref[...], hp_ref[...], dinv_ref[...], stats_ref, g_ref, bb_ref)
    if residual:
        hn = hn + res_ref[...]
    h_ref[...] = hn
    hpn_ref[...] = dinv_ref[...] * _dot(hn, wn_ref[...])


def _k3(agg, hp, dinv, stats, g, bb, res, wnext, residual):
    body = functools.partial(_k3_body, residual=residual)
    return pl.pallas_call(
        body,
        grid=(_NG,),
        in_specs=[
            pl.BlockSpec((_RB, _H), lambda i: (i, 0)),
            pl.BlockSpec((_RB, _H), lambda i: (i, 0)),
            pl.BlockSpec((_RB, 1), lambda i: (i, 0)),
            pl.BlockSpec((2, _H), lambda i: (0, 0)),
            pl.BlockSpec((1, _H), lambda i: (0, 0)),
            pl.BlockSpec((1, _H), lambda i: (0, 0)),
            pl.BlockSpec((_RB, _H), lambda i: (i, 0)),
            pl.BlockSpec((_H, _H), lambda i: (0, 0)),
        ],
        out_specs=[
            pl.BlockSpec((_RB, _H), lambda i: (i, 0)),
            pl.BlockSpec((_RB, _H), lambda i: (i, 0)),
        ],
        out_shape=[
            jax.ShapeDtypeStruct((_N, _H), jnp.float32),
            jax.ShapeDtypeStruct((_N, _H), jnp.float32),
        ],
    )(agg, hp, dinv, stats, g, bb, res, wnext)


def _k3f_body(agg_ref, hp_ref, dinv_ref, stats_ref, g_ref, bb_ref, res_ref,
              gfp_ref, wgfp_ref, bgf_ref, w1a_ref, w1b_ref, w1c_ref, b1_ref,
              w2_ref, b2_ref, w3_ref, b3_ref, q_ref, psum_ref, pmax_ref):
    i = pl.program_id(0)
    hn = _bn_relu(agg_ref[...], hp_ref[...], dinv_ref[...], stats_ref, g_ref, bb_ref)
    hn = hn + res_ref[...]

    @pl.when(i == 0)
    def _():
        psum_ref[...] = jnp.zeros_like(psum_ref)
        pmax_ref[...] = jnp.full_like(pmax_ref, -jnp.inf)

    psum_ref[...] += jnp.sum(hn, axis=0, keepdims=True)
    pmax_ref[...] = jnp.maximum(pmax_ref[...], jnp.max(hn, axis=0, keepdims=True))

    @pl.when(i == _NG - 1)
    def _():
        hm = psum_ref[...] / _N
        hx = pmax_ref[...]
        gfh = jnp.maximum(_dot(gfp_ref[...], wgfp_ref[...]) + bgf_ref[...], 0.0)
        z1 = jnp.maximum(_dot(hm, w1a_ref[...]) + _dot(hx, w1b_ref[...])
                         + _dot(gfh, w1c_ref[...]) + b1_ref[...], 0.0)
        z2 = jnp.maximum(_dot(z1, w2_ref[...]) + b2_ref[...], 0.0)
        q_ref[...] = _dot(z2, w3_ref[...]) + b3_ref[...]


def _k3f(agg, hp, dinv, stats, g, bb, res, gfp, wgfp, bgf,
         w1a, w1b, w1c, b1, w2, b2, w3, b3):
    full = lambda shape: pl.BlockSpec(shape, lambda i: (0, 0))
    return pl.pallas_call(
        _k3f_body,
        grid=(_NG,),
        in_specs=[
            pl.BlockSpec((_RB, _H), lambda i: (i, 0)),
            pl.BlockSpec((_RB, _H), lambda i: (i, 0)),
            pl.BlockSpec((_RB, 1), lambda i: (i, 0)),
            full((2, _H)),
            full((1, _H)),
            full((1, _H)),
            pl.BlockSpec((_RB, _H), lambda i: (i, 0)),
            full((1, 8)),
            full((8, _H)),
            full((1, _H)),
            full((_H, 2 * _H)),
            full((_H, 2 * _H)),
            full((_H, 2 * _H)),
            full((1, 2 * _H)),
            full((2 * _H, _H)),
            full((1, _H)),
            full((_H, _A)),
            full((1, _A)),
        ],
        out_specs=pl.BlockSpec((1, _A), lambda i: (0, 0)),
        out_shape=jax.ShapeDtypeStruct((1, _A), jnp.float32),
        scratch_shapes=[
            pltpu.VMEM((1, _H), jnp.float32),
            pltpu.VMEM((1, _H), jnp.float32),
        ],
    )(agg, hp, dinv, stats, g, bb, res, gfp, wgfp, bgf,
      w1a, w1b, w1c, b1, w2, b2, w3, b3)


_SEG = _seg_sc
_DEG = _deg_sc
_BKT = _bucketize_sc


def kernel(x, edge_index, global_features, w_emb, b_emb, w_gcn0, b_gcn0,
           w_gcn1, b_gcn1, w_gcn2, b_gcn2, bn_g0, bn_b0, bn_g1, bn_b1,
           bn_g2, bn_b2, w_gf, b_gf, w_q1, b_q1, w_q2, b_q2, w_q3, b_q3):
    src = edge_index[0]
    dst = edge_index[1]

    srcs, dstl, nch = _BKT(src, dst)
    deg = _DEG(dst)
    deg2d = deg.reshape(_DEGPAD, 1)[:_N]

    xpad = jnp.pad(x, ((0, 0), (0, 16 - x.shape[1])))
    wembp = jnp.pad(w_emb, ((0, 16 - w_emb.shape[0]), (0, 0)))
    gfp = jnp.pad(global_features, ((0, 0), (0, 8 - global_features.shape[1])))
    wgfp = jnp.pad(w_gf, ((0, 8 - w_gf.shape[0]), (0, 0)))

    hp0, dinv = _k1(xpad, deg2d, wembp, b_emb.reshape(1, _H), w_gcn0)

    agg0 = _SEG(hp0, srcs, dstl, nch)
    stats0 = _k2(agg0, hp0, dinv)
    h1, hp1 = _k3(agg0, hp0, dinv, stats0, bn_g0.reshape(1, _H),
                  bn_b0.reshape(1, _H), hp0, w_gcn1, residual=False)

    agg1 = _SEG(hp1, srcs, dstl, nch)
    stats1 = _k2(agg1, hp1, dinv)
    h2, hp2 = _k3(agg1, hp1, dinv, stats1, bn_g1.reshape(1, _H),
                  bn_b1.reshape(1, _H), h1, w_gcn2, residual=True)

    agg2 = _SEG(hp2, srcs, dstl, nch)
    stats2 = _k2(agg2, hp2, dinv)
    q = _k3f(agg2, hp2, dinv, stats2, bn_g2.reshape(1, _H),
             bn_b2.reshape(1, _H), h2, gfp, wgfp, b_gf.reshape(1, _H),
             w_q1[:_H], w_q1[_H:2 * _H], w_q1[2 * _H:], b_q1.reshape(1, 2 * _H),
             w_q2, b_q2.reshape(1, _H), w_q3, b_q3.reshape(1, _A))
    return q
